# Initial kernel scaffold; baseline (speedup 1.0000x reference)
import jax
import jax.numpy as jnp
from jax.experimental import pallas as pl


def kernel(per_atom_charge, per_system_total_charge, atomic_subsystem_indices):
    raise NotImplementedError("write your pallas kernel here")


# speedup vs baseline: 41.8753x; 41.8753x over previous
import functools

import jax
import jax.numpy as jnp
from jax import lax
from jax.experimental import pallas as pl
from jax.experimental.pallas import tpu as pltpu
from jax.experimental.pallas import tpu_sc as plsc

NC = 2
NS = 16
L = 16
RW = 128
NW = NC * NS


def _geometry(n, m):
    rows = -(-n // (NW * RW))
    n_pad = NW * rows * RW
    m_pad = (m // (NS * L) + 1) * NS * L
    return rows, n_pad, m_pad


def _make_k1(rows, m_pad):
    msl = m_pad // NS

    def body(idx_hbm, chg_hbm, psum_hbm, pcnt_hbm,
             ssum, scnt, idx_v, val_v, zf, zi, ones_v, sem):
        c = lax.axis_index("c")
        s = lax.axis_index("s")
        wid = s * NC + c
        cp_i = pltpu.async_copy(idx_hbm.at[wid], idx_v, sem)
        cp_v = pltpu.async_copy(chg_hbm.at[wid], val_v, sem)

        def zbody(i, carry):
            zf[pl.ds(i * L, L)] = jnp.zeros((L,), jnp.float32)
            zi[pl.ds(i * L, L)] = jnp.zeros((L,), jnp.int32)
            return carry

        lax.fori_loop(0, msl // L, zbody, 0)
        for i in range(RW // L):
            ones_v[pl.ds(i * L, L)] = jnp.ones((L,), jnp.int32)
        pltpu.sync_copy(zf, ssum.at[pl.ds(s * msl, msl)])
        pltpu.sync_copy(zi, scnt.at[pl.ds(s * msl, msl)])
        plsc.subcore_barrier()
        cp_i.wait()
        cp_v.wait()

        def sbody(j, carry):
            pltpu.sync_copy(val_v.at[j], ssum.at[idx_v.at[j]], add=True)
            pltpu.sync_copy(ones_v, scnt.at[idx_v.at[j]], add=True)
            return carry

        lax.fori_loop(0, rows, sbody, 0)
        plsc.subcore_barrier()
        pltpu.sync_copy(ssum.at[pl.ds(s * msl, msl)], zf)
        pltpu.sync_copy(zf, psum_hbm.at[pl.ds(c * m_pad + s * msl, msl)])
        pltpu.sync_copy(scnt.at[pl.ds(s * msl, msl)], zi)
        pltpu.sync_copy(zi, pcnt_hbm.at[pl.ds(c * m_pad + s * msl, msl)])

    return pl.kernel(
        body,
        out_type=[jax.ShapeDtypeStruct((NC * m_pad,), jnp.float32),
                  jax.ShapeDtypeStruct((NC * m_pad,), jnp.int32)],
        mesh=plsc.VectorSubcoreMesh(core_axis_name="c", subcore_axis_name="s",
                                    num_cores=NC, num_subcores=NS),
        scratch_types=[
            pltpu.VMEM_SHARED((m_pad,), jnp.float32),
            pltpu.VMEM_SHARED((m_pad,), jnp.int32),
            pltpu.VMEM((rows, RW), jnp.int32),
            pltpu.VMEM((rows, RW), jnp.float32),
            pltpu.VMEM((msl,), jnp.float32),
            pltpu.VMEM((msl,), jnp.int32),
            pltpu.VMEM((RW,), jnp.int32),
            pltpu.SemaphoreType.DMA,
        ],
    )


def _make_k2(rows, m_pad):
    msl = m_pad // NS

    def body(idx_hbm, chg_hbm, psum_hbm, pcnt_hbm, tot_hbm, out_hbm,
             scorr, idx_v, val_v, cor_v, s0v, s1v, n0v, n1v, totv, cbuf,
             sem_i, sem_v):
        c = lax.axis_index("c")
        s = lax.axis_index("s")
        wid = s * NC + c
        base = s * msl
        cp_i = pltpu.async_copy(idx_hbm.at[wid], idx_v, sem_i)
        cp_v = pltpu.async_copy(chg_hbm.at[wid], val_v, sem_v)
        pltpu.sync_copy(psum_hbm.at[pl.ds(base, msl)], s0v)
        pltpu.sync_copy(psum_hbm.at[pl.ds(m_pad + base, msl)], s1v)
        pltpu.sync_copy(pcnt_hbm.at[pl.ds(base, msl)], n0v)
        pltpu.sync_copy(pcnt_hbm.at[pl.ds(m_pad + base, msl)], n1v)
        pltpu.sync_copy(tot_hbm.at[pl.ds(base, msl)], totv)

        def cbody(i, carry):
            sl = pl.ds(i * L, L)
            seg_sum = s0v[sl] + s1v[sl]
            cnt = (n0v[sl] + n1v[sl]).astype(jnp.float32)
            cbuf[sl] = (totv[sl] - seg_sum) / cnt
            return carry

        lax.fori_loop(0, msl // L, cbody, 0)
        pltpu.sync_copy(cbuf, scorr.at[pl.ds(base, msl)])
        plsc.subcore_barrier()
        cp_i.wait()

        def gbody(j, carry):
            pltpu.sync_copy(scorr.at[idx_v.at[j]], cor_v.at[j])
            return carry

        lax.fori_loop(0, rows, gbody, 0)
        cp_v.wait()

        def abody(j, carry):
            for l in range(RW // L):
                sl = pl.ds(l * L, L)
                cor_v[j, sl] = cor_v[j, sl] + val_v[j, sl]
            return carry

        lax.fori_loop(0, rows, abody, 0)
        pltpu.sync_copy(cor_v, out_hbm.at[wid])

    return pl.kernel(
        body,
        out_type=jax.ShapeDtypeStruct((NW, rows, RW), jnp.float32),
        mesh=plsc.VectorSubcoreMesh(core_axis_name="c", subcore_axis_name="s",
                                    num_cores=NC, num_subcores=NS),
        scratch_types=[
            pltpu.VMEM_SHARED((m_pad,), jnp.float32),
            pltpu.VMEM((rows, RW), jnp.int32),
            pltpu.VMEM((rows, RW), jnp.float32),
            pltpu.VMEM((rows, RW), jnp.float32),
            pltpu.VMEM((msl,), jnp.float32),
            pltpu.VMEM((msl,), jnp.float32),
            pltpu.VMEM((msl,), jnp.int32),
            pltpu.VMEM((msl,), jnp.int32),
            pltpu.VMEM((msl,), jnp.float32),
            pltpu.VMEM((msl,), jnp.float32),
            pltpu.SemaphoreType.DMA,
            pltpu.SemaphoreType.DMA,
        ],
    )


def kernel(per_atom_charge, per_system_total_charge, atomic_subsystem_indices):
    n = per_atom_charge.shape[0]
    m = per_system_total_charge.shape[0]
    rows, n_pad, m_pad = _geometry(n, m)

    chg = per_atom_charge.reshape(-1).astype(jnp.float32)
    idx = atomic_subsystem_indices.astype(jnp.int32)
    tot = per_system_total_charge.reshape(-1).astype(jnp.float32)

    p = n_pad - n
    pad_idx = (jnp.arange(p, dtype=jnp.int32) % (m_pad - m)) + m
    idx_p = jnp.concatenate([idx, pad_idx]).reshape(NW, rows, RW)
    chg_p = jnp.concatenate([chg, jnp.zeros((p,), jnp.float32)]
                            ).reshape(NW, rows, RW)
    tot_p = jnp.concatenate([tot, jnp.zeros((m_pad - m,), jnp.float32)])

    psum, pcnt = _make_k1(rows, m_pad)(idx_p, chg_p)
    out = _make_k2(rows, m_pad)(idx_p, chg_p, psum, pcnt, tot_p)
    return out.reshape(-1)[:n][:, None]


# speedup vs baseline: 45.2205x; 1.0799x over previous
import functools

import jax
import jax.numpy as jnp
from jax import lax
from jax.experimental import pallas as pl
from jax.experimental.pallas import tpu as pltpu
from jax.experimental.pallas import tpu_sc as plsc

NC = 2
NS = 16
L = 16
RW = 128
NW = NC * NS


def _geometry(n, m):
    rows = -(-n // (NW * RW))
    m_pad = (m // (NS * L) + 1) * NS * L
    return rows, m_pad


def _make_k1(n, m, rows, m_pad):
    msl = m_pad // NS
    ch = rows * RW
    v_last = n - (NW - 1) * ch
    tail = ch - v_last
    pr = ((m_pad - m) // L) * L

    def body(idx_hbm, chg_hbm, psum_hbm, pcnt_hbm,
             ssum, scnt, idx_v, val_v, zf, zi, ones_v, sem_st):
        c = lax.axis_index("c")
        s = lax.axis_index("s")
        wid = s * NC + c
        base = wid * ch

        @pl.when(wid < NW - 1)
        def _():
            pltpu.async_copy(idx_hbm.at[pl.ds(base, ch)], idx_v, sem_st)
            pltpu.async_copy(chg_hbm.at[pl.ds(base, ch)], val_v, sem_st)

        @pl.when(wid == NW - 1)
        def _():
            pltpu.async_copy(idx_hbm.at[pl.ds(base, v_last)],
                             idx_v.at[pl.ds(0, v_last)], sem_st)
            pltpu.async_copy(chg_hbm.at[pl.ds(base, v_last)],
                             val_v.at[pl.ds(0, v_last)], sem_st)

        def zbody(i, carry):
            zf[pl.ds(i * L, L)] = jnp.zeros((L,), jnp.float32)
            zi[pl.ds(i * L, L)] = jnp.zeros((L,), jnp.int32)
            return carry

        lax.fori_loop(0, msl // L, zbody, 0)
        for i in range(RW // L):
            ones_v[pl.ds(i * L, L)] = jnp.ones((L,), jnp.int32)
        pltpu.sync_copy(zf, ssum.at[pl.ds(s * msl, msl)])
        pltpu.sync_copy(zi, scnt.at[pl.ds(s * msl, msl)])
        plsc.subcore_barrier()

        @pl.when(wid < NW - 1)
        def _():
            pltpu.make_async_copy(idx_hbm.at[pl.ds(base, ch)],
                                  idx_v, sem_st).wait()
            pltpu.make_async_copy(chg_hbm.at[pl.ds(base, ch)],
                                  val_v, sem_st).wait()

        @pl.when(wid == NW - 1)
        def _():
            pltpu.make_async_copy(idx_hbm.at[pl.ds(base, v_last)],
                                  idx_v.at[pl.ds(0, v_last)], sem_st).wait()
            pltpu.make_async_copy(chg_hbm.at[pl.ds(base, v_last)],
                                  val_v.at[pl.ds(0, v_last)], sem_st).wait()

            def tbody(i, carry):
                lanes = pl.ds(v_last + i * L, L)
                idx_v[lanes] = (m + lax.rem(i * L, pr)
                                + lax.iota(jnp.int32, L))
                val_v[lanes] = jnp.zeros((L,), jnp.float32)
                return carry

            lax.fori_loop(0, tail // L, tbody, 0)

        def sbody(j, carry):
            r = pl.ds(j * RW, RW)
            pltpu.sync_copy(val_v.at[r], ssum.at[idx_v.at[r]], add=True)
            pltpu.sync_copy(ones_v, scnt.at[idx_v.at[r]], add=True)
            return carry

        lax.fori_loop(0, rows, sbody, 0)
        plsc.subcore_barrier()
        pltpu.sync_copy(ssum.at[pl.ds(s * msl, msl)], zf)
        pltpu.sync_copy(zf, psum_hbm.at[pl.ds(c * m_pad + s * msl, msl)])
        pltpu.sync_copy(scnt.at[pl.ds(s * msl, msl)], zi)
        pltpu.sync_copy(zi, pcnt_hbm.at[pl.ds(c * m_pad + s * msl, msl)])

    return pl.kernel(
        body,
        out_type=[jax.ShapeDtypeStruct((NC * m_pad,), jnp.float32),
                  jax.ShapeDtypeStruct((NC * m_pad,), jnp.int32)],
        mesh=plsc.VectorSubcoreMesh(core_axis_name="c", subcore_axis_name="s",
                                    num_cores=NC, num_subcores=NS),
        scratch_types=[
            pltpu.VMEM_SHARED((m_pad,), jnp.float32),
            pltpu.VMEM_SHARED((m_pad,), jnp.int32),
            pltpu.VMEM((ch,), jnp.int32),
            pltpu.VMEM((ch,), jnp.float32),
            pltpu.VMEM((msl,), jnp.float32),
            pltpu.VMEM((msl,), jnp.int32),
            pltpu.VMEM((RW,), jnp.int32),
            pltpu.SemaphoreType.DMA,
        ],
    )


def _make_k2(n, m, rows, m_pad):
    msl = m_pad // NS
    ch = rows * RW
    v_last = n - (NW - 1) * ch
    tail = ch - v_last
    pr = ((m_pad - m) // L) * L
    m_tail = m - (NS - 1) * msl

    def body(idx_hbm, chg_hbm, psum_hbm, pcnt_hbm, tot_hbm, out_hbm,
             scorr, idx_v, val_v, cor_v, s0v, s1v, n0v, n1v, totv, cbuf,
             sem_st):
        c = lax.axis_index("c")
        s = lax.axis_index("s")
        wid = s * NC + c
        base = wid * ch
        bm = s * msl

        @pl.when(wid < NW - 1)
        def _():
            pltpu.async_copy(idx_hbm.at[pl.ds(base, ch)], idx_v, sem_st)
            pltpu.async_copy(chg_hbm.at[pl.ds(base, ch)], val_v, sem_st)

        @pl.when(wid == NW - 1)
        def _():
            pltpu.async_copy(idx_hbm.at[pl.ds(base, v_last)],
                             idx_v.at[pl.ds(0, v_last)], sem_st)
            pltpu.async_copy(chg_hbm.at[pl.ds(base, v_last)],
                             val_v.at[pl.ds(0, v_last)], sem_st)

        pltpu.sync_copy(psum_hbm.at[pl.ds(bm, msl)], s0v)
        pltpu.sync_copy(psum_hbm.at[pl.ds(m_pad + bm, msl)], s1v)
        pltpu.sync_copy(pcnt_hbm.at[pl.ds(bm, msl)], n0v)
        pltpu.sync_copy(pcnt_hbm.at[pl.ds(m_pad + bm, msl)], n1v)

        @pl.when(s < NS - 1)
        def _():
            pltpu.sync_copy(tot_hbm.at[pl.ds(bm, msl)], totv)

        @pl.when(s == NS - 1)
        def _():
            pltpu.sync_copy(tot_hbm.at[pl.ds(bm, m_tail)],
                            totv.at[pl.ds(0, m_tail)])

        def cbody(i, carry):
            sl = pl.ds(i * L, L)
            seg_sum = s0v[sl] + s1v[sl]
            cnt = (n0v[sl] + n1v[sl]).astype(jnp.float32)
            cbuf[sl] = (totv[sl] - seg_sum) / cnt
            return carry

        lax.fori_loop(0, msl // L, cbody, 0)
        pltpu.sync_copy(cbuf, scorr.at[pl.ds(bm, msl)])
        plsc.subcore_barrier()

        @pl.when(wid < NW - 1)
        def _():
            pltpu.make_async_copy(idx_hbm.at[pl.ds(base, ch)],
                                  idx_v, sem_st).wait()
            pltpu.make_async_copy(chg_hbm.at[pl.ds(base, ch)],
                                  val_v, sem_st).wait()

        @pl.when(wid == NW - 1)
        def _():
            pltpu.make_async_copy(idx_hbm.at[pl.ds(base, v_last)],
                                  idx_v.at[pl.ds(0, v_last)], sem_st).wait()
            pltpu.make_async_copy(chg_hbm.at[pl.ds(base, v_last)],
                                  val_v.at[pl.ds(0, v_last)], sem_st).wait()

            def tbody(i, carry):
                lanes = pl.ds(v_last + i * L, L)
                idx_v[lanes] = (m + lax.rem(i * L, pr)
                                + lax.iota(jnp.int32, L))
                val_v[lanes] = jnp.zeros((L,), jnp.float32)
                return carry

            lax.fori_loop(0, tail // L, tbody, 0)

        def gbody(j, carry):
            r = pl.ds(j * RW, RW)
            pltpu.sync_copy(scorr.at[idx_v.at[r]], cor_v.at[r])
            return carry

        lax.fori_loop(0, rows, gbody, 0)

        def abody(j, carry):
            for l in range(RW // L):
                sl = pl.ds(j * RW + l * L, L)
                cor_v[sl] = cor_v[sl] + val_v[sl]
            return carry

        lax.fori_loop(0, rows, abody, 0)

        @pl.when(wid < NW - 1)
        def _():
            pltpu.sync_copy(cor_v, out_hbm.at[pl.ds(base, ch)])

        @pl.when(wid == NW - 1)
        def _():
            pltpu.sync_copy(cor_v.at[pl.ds(0, v_last)],
                            out_hbm.at[pl.ds(base, v_last)])

    return pl.kernel(
        body,
        out_type=jax.ShapeDtypeStruct((n,), jnp.float32),
        mesh=plsc.VectorSubcoreMesh(core_axis_name="c", subcore_axis_name="s",
                                    num_cores=NC, num_subcores=NS),
        scratch_types=[
            pltpu.VMEM_SHARED((m_pad,), jnp.float32),
            pltpu.VMEM((ch,), jnp.int32),
            pltpu.VMEM((ch,), jnp.float32),
            pltpu.VMEM((ch,), jnp.float32),
            pltpu.VMEM((msl,), jnp.float32),
            pltpu.VMEM((msl,), jnp.float32),
            pltpu.VMEM((msl,), jnp.int32),
            pltpu.VMEM((msl,), jnp.int32),
            pltpu.VMEM((msl,), jnp.float32),
            pltpu.VMEM((msl,), jnp.float32),
            pltpu.SemaphoreType.DMA,
        ],
    )


def kernel(per_atom_charge, per_system_total_charge, atomic_subsystem_indices):
    n = per_atom_charge.shape[0]
    m = per_system_total_charge.shape[0]
    rows, m_pad = _geometry(n, m)

    chg = per_atom_charge.reshape(-1).astype(jnp.float32)
    idx = atomic_subsystem_indices.astype(jnp.int32)
    tot = per_system_total_charge.reshape(-1).astype(jnp.float32)

    psum, pcnt = _make_k1(n, m, rows, m_pad)(idx, chg)
    out = _make_k2(n, m, rows, m_pad)(idx, chg, psum, pcnt, tot)
    return out[:, None]


# speedup vs baseline: 53.3186x; 1.1791x over previous
import functools

import jax
import jax.numpy as jnp
from jax import lax
from jax.experimental import pallas as pl
from jax.experimental.pallas import tpu as pltpu
from jax.experimental.pallas import tpu_sc as plsc

NC = 2
NS = 16
L = 16
RW = 128
NW = NC * NS


def _geometry(n, m):
    rows = -(-n // (NW * RW))
    m_pad = (m // (NS * L) + 1) * NS * L
    return rows, m_pad


def _make_k1(n, m, rows, m_pad):
    msl = m_pad // NS
    ch = rows * RW
    v_last = n - (NW - 1) * ch
    tail = ch - v_last
    pr = ((m_pad - m) // L) * L

    def body(idx_hbm, chg_hbm, psum_hbm, pcnt_hbm,
             ssum, scnt, idx_v, val_v, zf, zi, ones_v, sem_st, sem_sc):
        c = lax.axis_index("c")
        s = lax.axis_index("s")
        wid = s * NC + c
        base = wid * ch

        @pl.when(wid < NW - 1)
        def _():
            pltpu.async_copy(idx_hbm.at[pl.ds(base, ch)], idx_v, sem_st)
            pltpu.async_copy(chg_hbm.at[pl.ds(base, ch)], val_v, sem_st)

        @pl.when(wid == NW - 1)
        def _():
            pltpu.async_copy(idx_hbm.at[pl.ds(base, v_last)],
                             idx_v.at[pl.ds(0, v_last)], sem_st)
            pltpu.async_copy(chg_hbm.at[pl.ds(base, v_last)],
                             val_v.at[pl.ds(0, v_last)], sem_st)

        def zbody(i, carry):
            zf[pl.ds(i * L, L)] = jnp.zeros((L,), jnp.float32)
            zi[pl.ds(i * L, L)] = jnp.zeros((L,), jnp.int32)
            return carry

        lax.fori_loop(0, msl // L, zbody, 0)
        for i in range(RW // L):
            ones_v[pl.ds(i * L, L)] = jnp.ones((L,), jnp.int32)
        pltpu.sync_copy(zf, ssum.at[pl.ds(s * msl, msl)])
        pltpu.sync_copy(zi, scnt.at[pl.ds(s * msl, msl)])
        plsc.subcore_barrier()

        @pl.when(wid < NW - 1)
        def _():
            pltpu.make_async_copy(idx_hbm.at[pl.ds(base, ch)],
                                  idx_v, sem_st).wait()
            pltpu.make_async_copy(chg_hbm.at[pl.ds(base, ch)],
                                  val_v, sem_st).wait()

        @pl.when(wid == NW - 1)
        def _():
            pltpu.make_async_copy(idx_hbm.at[pl.ds(base, v_last)],
                                  idx_v.at[pl.ds(0, v_last)], sem_st).wait()
            pltpu.make_async_copy(chg_hbm.at[pl.ds(base, v_last)],
                                  val_v.at[pl.ds(0, v_last)], sem_st).wait()

            def tbody(i, carry):
                lanes = pl.ds(v_last + i * L, L)
                idx_v[lanes] = (m + lax.rem(i * L, pr)
                                + lax.iota(jnp.int32, L))
                val_v[lanes] = jnp.zeros((L,), jnp.float32)
                return carry

            lax.fori_loop(0, tail // L, tbody, 0)

        ck = 7 if rows % 7 == 0 else 1
        nck = rows // ck

        def sbody(cidx, carry):
            for r in range(ck):
                rr = pl.ds((cidx * ck + r) * RW, RW)
                pltpu.async_copy(val_v.at[rr], ssum.at[idx_v.at[rr]],
                                 sem_sc, add=True)
                pltpu.async_copy(ones_v, scnt.at[idx_v.at[rr]],
                                 sem_sc, add=True)

            @pl.when(cidx > 0)
            def _():
                pltpu.make_async_copy(chg_hbm.at[pl.ds(0, 2 * ck * RW)],
                                      val_v.at[pl.ds(0, 2 * ck * RW)],
                                      sem_sc).wait()

            return carry

        lax.fori_loop(0, nck, sbody, 0)
        pltpu.make_async_copy(chg_hbm.at[pl.ds(0, 2 * ck * RW)],
                              val_v.at[pl.ds(0, 2 * ck * RW)], sem_sc).wait()
        plsc.subcore_barrier()
        pltpu.sync_copy(ssum.at[pl.ds(s * msl, msl)], zf)
        pltpu.sync_copy(zf, psum_hbm.at[pl.ds(c * m_pad + s * msl, msl)])
        pltpu.sync_copy(scnt.at[pl.ds(s * msl, msl)], zi)
        pltpu.sync_copy(zi, pcnt_hbm.at[pl.ds(c * m_pad + s * msl, msl)])

    return pl.kernel(
        body,
        out_type=[jax.ShapeDtypeStruct((NC * m_pad,), jnp.float32),
                  jax.ShapeDtypeStruct((NC * m_pad,), jnp.int32)],
        mesh=plsc.VectorSubcoreMesh(core_axis_name="c", subcore_axis_name="s",
                                    num_cores=NC, num_subcores=NS),
        scratch_types=[
            pltpu.VMEM_SHARED((m_pad,), jnp.float32),
            pltpu.VMEM_SHARED((m_pad,), jnp.int32),
            pltpu.VMEM((ch,), jnp.int32),
            pltpu.VMEM((ch,), jnp.float32),
            pltpu.VMEM((msl,), jnp.float32),
            pltpu.VMEM((msl,), jnp.int32),
            pltpu.VMEM((RW,), jnp.int32),
            pltpu.SemaphoreType.DMA,
            pltpu.SemaphoreType.DMA,
        ],
    )


def _make_k2(n, m, rows, m_pad):
    msl = m_pad // NS
    ch = rows * RW
    v_last = n - (NW - 1) * ch
    tail = ch - v_last
    pr = ((m_pad - m) // L) * L
    m_tail = m - (NS - 1) * msl

    def body(idx_hbm, chg_hbm, psum_hbm, pcnt_hbm, tot_hbm, out_hbm,
             scorr, idx_v, val_v, cor_v, s0v, s1v, n0v, n1v, totv, cbuf,
             sem_st, sem_g):
        c = lax.axis_index("c")
        s = lax.axis_index("s")
        wid = s * NC + c
        base = wid * ch
        bm = s * msl

        @pl.when(wid < NW - 1)
        def _():
            pltpu.async_copy(idx_hbm.at[pl.ds(base, ch)], idx_v, sem_st)
            pltpu.async_copy(chg_hbm.at[pl.ds(base, ch)], val_v, sem_st)

        @pl.when(wid == NW - 1)
        def _():
            pltpu.async_copy(idx_hbm.at[pl.ds(base, v_last)],
                             idx_v.at[pl.ds(0, v_last)], sem_st)
            pltpu.async_copy(chg_hbm.at[pl.ds(base, v_last)],
                             val_v.at[pl.ds(0, v_last)], sem_st)

        pltpu.sync_copy(psum_hbm.at[pl.ds(bm, msl)], s0v)
        pltpu.sync_copy(psum_hbm.at[pl.ds(m_pad + bm, msl)], s1v)
        pltpu.sync_copy(pcnt_hbm.at[pl.ds(bm, msl)], n0v)
        pltpu.sync_copy(pcnt_hbm.at[pl.ds(m_pad + bm, msl)], n1v)

        @pl.when(s < NS - 1)
        def _():
            pltpu.sync_copy(tot_hbm.at[pl.ds(bm, msl)], totv)

        @pl.when(s == NS - 1)
        def _():
            pltpu.sync_copy(tot_hbm.at[pl.ds(bm, m_tail)],
                            totv.at[pl.ds(0, m_tail)])

        def cbody(i, carry):
            sl = pl.ds(i * L, L)
            seg_sum = s0v[sl] + s1v[sl]
            cnt = (n0v[sl] + n1v[sl]).astype(jnp.float32)
            cbuf[sl] = (totv[sl] - seg_sum) / cnt
            return carry

        lax.fori_loop(0, msl // L, cbody, 0)
        pltpu.sync_copy(cbuf, scorr.at[pl.ds(bm, msl)])
        plsc.subcore_barrier()

        @pl.when(wid < NW - 1)
        def _():
            pltpu.make_async_copy(idx_hbm.at[pl.ds(base, ch)],
                                  idx_v, sem_st).wait()
            pltpu.make_async_copy(chg_hbm.at[pl.ds(base, ch)],
                                  val_v, sem_st).wait()

        @pl.when(wid == NW - 1)
        def _():
            pltpu.make_async_copy(idx_hbm.at[pl.ds(base, v_last)],
                                  idx_v.at[pl.ds(0, v_last)], sem_st).wait()
            pltpu.make_async_copy(chg_hbm.at[pl.ds(base, v_last)],
                                  val_v.at[pl.ds(0, v_last)], sem_st).wait()

            def tbody(i, carry):
                lanes = pl.ds(v_last + i * L, L)
                idx_v[lanes] = (m + lax.rem(i * L, pr)
                                + lax.iota(jnp.int32, L))
                val_v[lanes] = jnp.zeros((L,), jnp.float32)
                return carry

            lax.fori_loop(0, tail // L, tbody, 0)

        ck = 7 if rows % 7 == 0 else 1
        nck = rows // ck

        def fire(cidx):
            for r in range(ck):
                rr = pl.ds((cidx * ck + r) * RW, RW)
                pltpu.async_copy(scorr.at[idx_v.at[rr]], cor_v.at[rr], sem_g)

        def drain_one():
            pltpu.make_async_copy(chg_hbm.at[pl.ds(0, ck * RW)],
                                  cor_v.at[pl.ds(0, ck * RW)], sem_g).wait()

        def add_chunk(cidx):
            for r in range(ck):
                for l in range(RW // L):
                    sl = pl.ds((cidx * ck + r) * RW + l * L, L)
                    cor_v[sl] = cor_v[sl] + val_v[sl]

        fire(0)

        def gbody(cidx, carry):
            fire(cidx + 1)
            drain_one()
            add_chunk(cidx)
            return carry

        lax.fori_loop(0, nck - 1, gbody, 0)
        drain_one()
        add_chunk(nck - 1)

        @pl.when(wid < NW - 1)
        def _():
            pltpu.sync_copy(cor_v, out_hbm.at[pl.ds(base, ch)])

        @pl.when(wid == NW - 1)
        def _():
            pltpu.sync_copy(cor_v.at[pl.ds(0, v_last)],
                            out_hbm.at[pl.ds(base, v_last)])

    return pl.kernel(
        body,
        out_type=jax.ShapeDtypeStruct((n,), jnp.float32),
        mesh=plsc.VectorSubcoreMesh(core_axis_name="c", subcore_axis_name="s",
                                    num_cores=NC, num_subcores=NS),
        scratch_types=[
            pltpu.VMEM_SHARED((m_pad,), jnp.float32),
            pltpu.VMEM((ch,), jnp.int32),
            pltpu.VMEM((ch,), jnp.float32),
            pltpu.VMEM((ch,), jnp.float32),
            pltpu.VMEM((msl,), jnp.float32),
            pltpu.VMEM((msl,), jnp.float32),
            pltpu.VMEM((msl,), jnp.int32),
            pltpu.VMEM((msl,), jnp.int32),
            pltpu.VMEM((msl,), jnp.float32),
            pltpu.VMEM((msl,), jnp.float32),
            pltpu.SemaphoreType.DMA,
            pltpu.SemaphoreType.DMA,
        ],
    )


def kernel(per_atom_charge, per_system_total_charge, atomic_subsystem_indices):
    n = per_atom_charge.shape[0]
    m = per_system_total_charge.shape[0]
    rows, m_pad = _geometry(n, m)

    chg = per_atom_charge.reshape(-1).astype(jnp.float32)
    idx = atomic_subsystem_indices.astype(jnp.int32)
    tot = per_system_total_charge.reshape(-1).astype(jnp.float32)

    psum, pcnt = _make_k1(n, m, rows, m_pad)(idx, chg)
    out = _make_k2(n, m, rows, m_pad)(idx, chg, psum, pcnt, tot)
    return out[:, None]


# speedup vs baseline: 56.0323x; 1.0509x over previous
import functools

import jax
import jax.numpy as jnp
from jax import lax
from jax.experimental import pallas as pl
from jax.experimental.pallas import tpu as pltpu
from jax.experimental.pallas import tpu_sc as plsc

NC = 2
NS = 16
L = 16
RW = 128
NW = NC * NS


def _geometry(n, m):
    rows = -(-n // (NW * RW))
    m_pad = (m // (NS * L) + 1) * NS * L
    return rows, m_pad


def _make_k1(n, m, rows, m_pad):
    msl = m_pad // NS
    ch = rows * RW
    v_last = n - (NW - 1) * ch
    tail = ch - v_last
    pr = ((m_pad - m) // L) * L

    def body(idx_hbm, chg_hbm, psum_hbm, pcnt_hbm,
             ssum, scnt, idx_v, val_v, zf, zi, ones_v, sem_st, sem_sc):
        c = lax.axis_index("c")
        s = lax.axis_index("s")
        wid = s * NC + c
        base = wid * ch

        @pl.when(wid < NW - 1)
        def _():
            pltpu.async_copy(idx_hbm.at[pl.ds(base, ch)], idx_v, sem_st)
            pltpu.async_copy(chg_hbm.at[pl.ds(base, ch)], val_v, sem_st)

        @pl.when(wid == NW - 1)
        def _():
            pltpu.async_copy(idx_hbm.at[pl.ds(base, v_last)],
                             idx_v.at[pl.ds(0, v_last)], sem_st)
            pltpu.async_copy(chg_hbm.at[pl.ds(base, v_last)],
                             val_v.at[pl.ds(0, v_last)], sem_st)

        def zbody(i, carry):
            zf[pl.ds(i * L, L)] = jnp.zeros((L,), jnp.float32)
            zi[pl.ds(i * L, L)] = jnp.zeros((L,), jnp.int32)
            return carry

        lax.fori_loop(0, msl // L, zbody, 0)
        for i in range(RW // L):
            ones_v[pl.ds(i * L, L)] = jnp.ones((L,), jnp.int32)
        pltpu.sync_copy(zf, ssum.at[pl.ds(s * msl, msl)])
        pltpu.sync_copy(zi, scnt.at[pl.ds(s * msl, msl)])
        plsc.subcore_barrier()

        @pl.when(wid < NW - 1)
        def _():
            pltpu.make_async_copy(idx_hbm.at[pl.ds(base, ch)],
                                  idx_v, sem_st).wait()
            pltpu.make_async_copy(chg_hbm.at[pl.ds(base, ch)],
                                  val_v, sem_st).wait()

        @pl.when(wid == NW - 1)
        def _():
            pltpu.make_async_copy(idx_hbm.at[pl.ds(base, v_last)],
                                  idx_v.at[pl.ds(0, v_last)], sem_st).wait()
            pltpu.make_async_copy(chg_hbm.at[pl.ds(base, v_last)],
                                  val_v.at[pl.ds(0, v_last)], sem_st).wait()

            def tbody(i, carry):
                lanes = pl.ds(v_last + i * L, L)
                idx_v[lanes] = (m + lax.rem(i * L, pr)
                                + lax.iota(jnp.int32, L))
                val_v[lanes] = jnp.zeros((L,), jnp.float32)
                return carry

            lax.fori_loop(0, tail // L, tbody, 0)

        ck = 7 if rows % 7 == 0 else 1
        nck = rows // ck

        def sbody(cidx, carry):
            for r in range(ck):
                rr = pl.ds((cidx * ck + r) * RW, RW)
                pltpu.async_copy(val_v.at[rr], ssum.at[idx_v.at[rr]],
                                 sem_sc, add=True)
                pltpu.async_copy(ones_v, scnt.at[idx_v.at[rr]],
                                 sem_sc, add=True)

            @pl.when(cidx > 0)
            def _():
                pltpu.make_async_copy(chg_hbm.at[pl.ds(0, 2 * ck * RW)],
                                      val_v.at[pl.ds(0, 2 * ck * RW)],
                                      sem_sc).wait()

            return carry

        lax.fori_loop(0, nck, sbody, 0)
        pltpu.make_async_copy(chg_hbm.at[pl.ds(0, 2 * ck * RW)],
                              val_v.at[pl.ds(0, 2 * ck * RW)], sem_sc).wait()
        plsc.subcore_barrier()
        pltpu.sync_copy(ssum.at[pl.ds(s * msl, msl)], zf)
        pltpu.sync_copy(zf, psum_hbm.at[pl.ds(c * m_pad + s * msl, msl)])
        pltpu.sync_copy(scnt.at[pl.ds(s * msl, msl)], zi)
        pltpu.sync_copy(zi, pcnt_hbm.at[pl.ds(c * m_pad + s * msl, msl)])

    return pl.kernel(
        body,
        out_type=[jax.ShapeDtypeStruct((NC * m_pad,), jnp.float32),
                  jax.ShapeDtypeStruct((NC * m_pad,), jnp.int32)],
        mesh=plsc.VectorSubcoreMesh(core_axis_name="c", subcore_axis_name="s",
                                    num_cores=NC, num_subcores=NS),
        scratch_types=[
            pltpu.VMEM_SHARED((m_pad,), jnp.float32),
            pltpu.VMEM_SHARED((m_pad,), jnp.int32),
            pltpu.VMEM((ch,), jnp.int32),
            pltpu.VMEM((ch,), jnp.float32),
            pltpu.VMEM((msl,), jnp.float32),
            pltpu.VMEM((msl,), jnp.int32),
            pltpu.VMEM((RW,), jnp.int32),
            pltpu.SemaphoreType.DMA,
            pltpu.SemaphoreType.DMA,
        ],
    )


def _make_k2(n, m, rows, m_pad):
    msl = m_pad // NS
    ch = rows * RW
    v_last = n - (NW - 1) * ch
    tail = ch - v_last
    pr = ((m_pad - m) // L) * L
    m_tail = m - (NS - 1) * msl
    CW = 1024

    def body(idx_hbm, chg_hbm, psum_hbm, pcnt_hbm, tot_hbm, out_hbm,
             scorr, idx_v, val_v, win_v, s0v, s1v, n0v, n1v, totv, cbuf,
             sem_st):
        c = lax.axis_index("c")
        s = lax.axis_index("s")
        wid = s * NC + c
        base = wid * ch
        bm = s * msl

        @pl.when(wid < NW - 1)
        def _():
            pltpu.async_copy(idx_hbm.at[pl.ds(base, ch)], idx_v, sem_st)
            pltpu.async_copy(chg_hbm.at[pl.ds(base, ch)], val_v, sem_st)

        @pl.when(wid == NW - 1)
        def _():
            pltpu.async_copy(idx_hbm.at[pl.ds(base, v_last)],
                             idx_v.at[pl.ds(0, v_last)], sem_st)
            pltpu.async_copy(chg_hbm.at[pl.ds(base, v_last)],
                             val_v.at[pl.ds(0, v_last)], sem_st)

        pltpu.sync_copy(psum_hbm.at[pl.ds(bm, msl)], s0v)
        pltpu.sync_copy(psum_hbm.at[pl.ds(m_pad + bm, msl)], s1v)
        pltpu.sync_copy(pcnt_hbm.at[pl.ds(bm, msl)], n0v)
        pltpu.sync_copy(pcnt_hbm.at[pl.ds(m_pad + bm, msl)], n1v)

        @pl.when(s < NS - 1)
        def _():
            pltpu.sync_copy(tot_hbm.at[pl.ds(bm, msl)], totv)

        @pl.when(s == NS - 1)
        def _():
            pltpu.sync_copy(tot_hbm.at[pl.ds(bm, m_tail)],
                            totv.at[pl.ds(0, m_tail)])

        def cbody(i, carry):
            sl = pl.ds(i * L, L)
            seg_sum = s0v[sl] + s1v[sl]
            cnt = (n0v[sl] + n1v[sl]).astype(jnp.float32)
            cbuf[sl] = (totv[sl] - seg_sum) / cnt
            return carry

        lax.fori_loop(0, msl // L, cbody, 0)
        pltpu.sync_copy(cbuf, scorr.at[pl.ds(bm, msl)])
        plsc.subcore_barrier()

        @pl.when(wid < NW - 1)
        def _():
            pltpu.make_async_copy(idx_hbm.at[pl.ds(base, ch)],
                                  idx_v, sem_st).wait()
            pltpu.make_async_copy(chg_hbm.at[pl.ds(base, ch)],
                                  val_v, sem_st).wait()

        @pl.when(wid == NW - 1)
        def _():
            pltpu.make_async_copy(idx_hbm.at[pl.ds(base, v_last)],
                                  idx_v.at[pl.ds(0, v_last)], sem_st).wait()
            pltpu.make_async_copy(chg_hbm.at[pl.ds(base, v_last)],
                                  val_v.at[pl.ds(0, v_last)], sem_st).wait()

            def tbody(i, carry):
                lanes = pl.ds(v_last + i * L, L)
                idx_v[lanes] = (m + lax.rem(i * L, pr)
                                + lax.iota(jnp.int32, L))
                val_v[lanes] = jnp.zeros((L,), jnp.float32)
                return carry

            lax.fori_loop(0, tail // L, tbody, 0)

        ve = jnp.where(wid == NW - 1, v_last, ch)
        lo8 = (idx_v[pl.ds(0, L)][0] // 8) * 8
        hi = idx_v[pl.ds(ve - L, L)][L - 1]
        width = hi - lo8 + 1
        ncw = (width + CW - 1) // CW
        start_cap = ((m_pad - CW) // 8) * 8

        def wbody(k, carry):
            st = jnp.minimum(lo8 + k * CW, start_cap)
            pltpu.sync_copy(scorr.at[pl.ds(st, CW)],
                            win_v.at[pl.ds(st - lo8, CW)])
            return carry

        lax.fori_loop(0, ncw, wbody, 0)

        lo_v = jnp.full((L,), lo8, jnp.int32)

        def gbody(i, carry):
            sl = pl.ds(i * L, L)
            off = idx_v[sl] - lo_v
            g = plsc.load_gather(win_v, [off])
            val_v[sl] = val_v[sl] + g
            return carry

        lax.fori_loop(0, rows * (RW // L), gbody, 0)

        @pl.when(wid < NW - 1)
        def _():
            pltpu.sync_copy(val_v, out_hbm.at[pl.ds(base, ch)])

        @pl.when(wid == NW - 1)
        def _():
            pltpu.sync_copy(val_v.at[pl.ds(0, v_last)],
                            out_hbm.at[pl.ds(base, v_last)])

    return pl.kernel(
        body,
        out_type=jax.ShapeDtypeStruct((n,), jnp.float32),
        mesh=plsc.VectorSubcoreMesh(core_axis_name="c", subcore_axis_name="s",
                                    num_cores=NC, num_subcores=NS),
        compiler_params=pltpu.CompilerParams(needs_layout_passes=False),
        scratch_types=[
            pltpu.VMEM_SHARED((m_pad,), jnp.float32),
            pltpu.VMEM((ch,), jnp.int32),
            pltpu.VMEM((ch,), jnp.float32),
            pltpu.VMEM((m_pad,), jnp.float32),
            pltpu.VMEM((msl,), jnp.float32),
            pltpu.VMEM((msl,), jnp.float32),
            pltpu.VMEM((msl,), jnp.int32),
            pltpu.VMEM((msl,), jnp.int32),
            pltpu.VMEM((msl,), jnp.float32),
            pltpu.VMEM((msl,), jnp.float32),
            pltpu.SemaphoreType.DMA,
        ],
    )


def kernel(per_atom_charge, per_system_total_charge, atomic_subsystem_indices):
    n = per_atom_charge.shape[0]
    m = per_system_total_charge.shape[0]
    rows, m_pad = _geometry(n, m)

    chg = per_atom_charge.reshape(-1).astype(jnp.float32)
    idx = atomic_subsystem_indices.astype(jnp.int32)
    tot = per_system_total_charge.reshape(-1).astype(jnp.float32)

    psum, pcnt = _make_k1(n, m, rows, m_pad)(idx, chg)
    out = _make_k2(n, m, rows, m_pad)(idx, chg, psum, pcnt, tot)
    return out[:, None]
